# Initial kernel scaffold; baseline (speedup 1.0000x reference)
import jax
import jax.numpy as jnp
from jax.experimental import pallas as pl


def kernel(pos, neg, labels, entity_emb, relation_emb):
    raise NotImplementedError("write your pallas kernel here")


# speedup vs baseline: 6.0377x; 6.0377x over previous
import functools

import jax
import jax.numpy as jnp
from jax import lax
from jax.experimental import pallas as pl
from jax.experimental.pallas import tpu as pltpu
from jax.experimental.pallas import tpu_sc as plsc

_NUM_CORES = 2
_NUM_SUBCORES = 16
_NUM_WORKERS = _NUM_CORES * _NUM_SUBCORES
_BLK = 128
_MARGIN = 1.0


def _make_sc_kernel(batch, num_neg, dim):
    assert batch % (_NUM_WORKERS * _BLK) == 0
    assert _BLK % num_neg == 0
    assert dim % 16 == 0
    bpw = batch // _NUM_WORKERS
    neg_blocks = bpw * num_neg // _BLK
    pos_blocks = bpw // _BLK
    b_per_neg_blk = _BLK // num_neg
    nchunk = dim // 16

    mesh = plsc.VectorSubcoreMesh(
        core_axis_name="c", subcore_axis_name="s",
        num_cores=_NUM_CORES, num_subcores=_NUM_SUBCORES)

    @functools.partial(
        pl.kernel,
        out_type=jax.ShapeDtypeStruct((_NUM_CORES, 16), jnp.float32),
        mesh=mesh,
        compiler_params=pltpu.CompilerParams(needs_layout_passes=False),
        scratch_types=[
            pltpu.VMEM((_BLK,), jnp.int32),
            pltpu.VMEM((_BLK,), jnp.int32),
            pltpu.VMEM((_BLK,), jnp.int32),
            pltpu.VMEM((_BLK, dim), jnp.float32),
            pltpu.VMEM((_BLK, dim), jnp.float32),
            pltpu.VMEM((_BLK, dim), jnp.float32),
            pltpu.VMEM((bpw * 16,), jnp.float32),
            pltpu.VMEM_SHARED((16,), jnp.float32),
            pltpu.VMEM((16,), jnp.float32),
            pltpu.SemaphoreType.DMA,
            pltpu.SemaphoreType.DMA,
            pltpu.SemaphoreType.DMA,
        ],
    )
    def transe_sc(ph, pr, pt, nh, nr, nt, ent, rel, out,
                  h_idx, r_idx, t_idx, h_rows, r_rows, t_rows,
                  comb, shared, vtmp, sem_h, sem_r, sem_t):
        c = lax.axis_index("c")
        s = lax.axis_index("s")
        wid = s * _NUM_CORES + c
        base_b = wid * bpw

        def fetch(h_src, r_src, t_src, off):
            pltpu.sync_copy(h_src.at[pl.ds(off, _BLK)], h_idx)
            pltpu.sync_copy(r_src.at[pl.ds(off, _BLK)], r_idx)
            pltpu.sync_copy(t_src.at[pl.ds(off, _BLK)], t_idx)
            dh = pltpu.async_copy(ent.at[h_idx], h_rows, sem_h)
            dr = pltpu.async_copy(rel.at[r_idx], r_rows, sem_r)
            dt = pltpu.async_copy(ent.at[t_idx], t_rows, sem_t)
            dh.wait()
            dr.wait()
            dt.wait()

        def row_abs_sum(row, acc):
            for cc in range(nchunk):
                sl = pl.ds(cc * 16, 16)
                acc = acc + jnp.abs(h_rows[row, sl] + r_rows[row, sl]
                                    - t_rows[row, sl])
            return acc

        def neg_block(blk, carry):
            fetch(nh, nr, nt, (base_b + blk * b_per_neg_blk) * num_neg)

            def per_b(bl, inner):
                def per_j(j, acc):
                    return row_abs_sum(bl * num_neg + j, acc)
                acc = lax.fori_loop(0, num_neg, per_j,
                                    jnp.zeros((16,), jnp.float32))
                comb[pl.ds((blk * b_per_neg_blk + bl) * 16, 16)] = acc
                return inner
            return lax.fori_loop(0, b_per_neg_blk, per_b, carry)

        lax.fori_loop(0, neg_blocks, neg_block, jnp.int32(0))

        def pos_block(blk, total):
            fetch(ph, pr, pt, base_b + blk * _BLK)

            def per_i(i, tot):
                acc = row_abs_sum(i, jnp.zeros((16,), jnp.float32))
                nvec = comb[pl.ds((blk * _BLK + i) * 16, 16)]
                sval = jnp.sum(acc - nvec) * (1.0 / dim) + _MARGIN
                return tot + jnp.maximum(sval, 0.0)
            return lax.fori_loop(0, _BLK, per_i, total)

        total = lax.fori_loop(0, pos_blocks, pos_block, jnp.float32(0.0))

        @pl.when(s == 0)
        def _():
            vtmp[...] = jnp.zeros((16,), jnp.float32)
            pltpu.sync_copy(vtmp, shared)
        plsc.subcore_barrier()
        vtmp[...] = jnp.full((16,), total, jnp.float32)
        pltpu.sync_copy(vtmp, shared.at[lax.iota(jnp.int32, 16)], add=True)
        plsc.subcore_barrier()

        @pl.when(s == 0)
        def _():
            pltpu.sync_copy(shared, vtmp)
            pltpu.sync_copy(vtmp, out.at[c])

    return transe_sc


def kernel(pos, neg, labels, entity_emb, relation_emb):
    del labels
    batch, num_neg, _ = neg.shape
    dim = entity_emb.shape[1]
    pos = pos.astype(jnp.int32)
    negf = neg.reshape(-1, 3).astype(jnp.int32)
    sc = _make_sc_kernel(batch, num_neg, dim)
    out = sc(pos[:, 0], pos[:, 1], pos[:, 2],
             negf[:, 0], negf[:, 1], negf[:, 2],
             entity_emb, relation_emb)
    return out[0, 0] + out[1, 0]
